# baseline (speedup 1.0000x reference)
import jax
import jax.numpy as jnp
from jax import lax
from jax.experimental import pallas as pl
from jax.experimental.pallas import tpu as pltpu
from jax.experimental.pallas import tpu_sc as plsc

N = 10000
H = 128
E = 320000
NC = 2
NS = 16
NW = NC * NS
CH = 128
EPW = 10240
PE = EPW * NW
CPW = EPW // CH
ROWS = 10240
RPT = ROWS // NS
CHD = 512
CPWD = EPW // CHD
BR = 2000

_mesh = plsc.VectorSubcoreMesh(core_axis_name="c", subcore_axis_name="s")


def _deg_body(d2_hbm, out_hbm, deg_sh, idxb, ones_v, zb_v, isem):
    cid = lax.axis_index("c")
    sid = lax.axis_index("s")
    wid = sid * NC + cid

    @pl.loop(0, RPT // 16)
    def _(i):
        zb_v[pl.ds(i * 16, 16)] = jnp.zeros((16,), jnp.float32)

    pltpu.sync_copy(zb_v, deg_sh.at[pl.ds(sid * RPT, RPT)])

    @pl.loop(0, CH // 16)
    def _(i):
        ones_v[pl.ds(i * 16, 16)] = jnp.full((16,), 1.0, jnp.float32)

    plsc.subcore_barrier()
    base = wid * (EPW // CH)
    rpc = CHD // CH
    pltpu.async_copy(d2_hbm.at[pl.ds(base, rpc)], idxb[0], isem[0])

    @pl.loop(0, CPWD, step=2)
    def _(c):
        for b in (0, 1):
            cc = c + b

            @pl.when(cc + 1 < CPWD)
            def _():
                pltpu.async_copy(
                    d2_hbm.at[pl.ds(base + (cc + 1) * rpc, rpc)],
                    idxb[b ^ 1], isem[b ^ 1])

            pltpu.make_async_copy(d2_hbm.at[pl.ds(0, rpc)], idxb[b],
                                  isem[b]).wait()
            for j in range(rpc):
                pltpu.sync_copy(ones_v, deg_sh.at[idxb[b].at[j]], add=True)

    plsc.subcore_barrier()
    pltpu.sync_copy(deg_sh.at[pl.ds(sid * RPT, RPT)],
                    out_hbm.at[cid].at[pl.ds(sid * RPT, RPT)])


_deg_kernel = pl.kernel(
    _deg_body,
    out_type=jax.ShapeDtypeStruct((NC, ROWS), jnp.float32),
    mesh=_mesh,
    scratch_types=[
        pltpu.VMEM_SHARED((ROWS,), jnp.float32),
        [pltpu.VMEM((CHD // CH, CH), jnp.int32)] * 2,
        pltpu.VMEM((CH,), jnp.float32),
        pltpu.VMEM((RPT,), jnp.float32),
        [pltpu.SemaphoreType.DMA] * 2,
    ],
)


def _agg_body(h_hbm, pk_hbm, out_hbm, acc_sh, idxb, rows_v, gsem, isem, ssem):
    cid = lax.axis_index("c")
    sid = lax.axis_index("s")
    wid = sid * NC + cid

    @pl.loop(0, CH)
    def _(r):
        @pl.loop(0, H // 16)
        def _(j):
            rows_v[0].at[r][pl.ds(j * 16, 16)] = jnp.zeros((16,), jnp.float32)

    @pl.loop(0, RPT // CH)
    def _(i):
        pltpu.sync_copy(rows_v[0], acc_sh.at[pl.ds(sid * RPT + i * CH, CH)])

    plsc.subcore_barrier()
    base = wid * CPW

    for k in range(3):
        pltpu.async_copy(pk_hbm.at[base + k], idxb[k], isem[k])
    pltpu.make_async_copy(pk_hbm.at[0], idxb[0], isem[0]).wait()
    pltpu.async_copy(h_hbm.at[idxb[0].at[0]], rows_v[0], gsem[0])

    @pl.loop(0, CPW, step=8)
    def _(c):
        for u in range(8):
            cc = c + u
            s2 = u % 2
            s8 = u
            n2 = (u + 1) % 2
            n8 = (u + 1) % 8

            @pl.when((cc >= 1) & (cc + 1 < CPW))
            def _():
                pltpu.make_async_copy(rows_v[n2],
                                      acc_sh.at[idxb[0].at[1]],
                                      ssem[n2]).wait()

            @pl.when(cc + 1 < CPW)
            def _():
                pltpu.make_async_copy(pk_hbm.at[0], idxb[n8],
                                      isem[n8]).wait()
                pltpu.async_copy(h_hbm.at[idxb[n8].at[0]], rows_v[n2],
                                 gsem[n2])

            @pl.when(cc + 3 < CPW)
            def _():
                pltpu.async_copy(pk_hbm.at[base + cc + 3], idxb[(u + 3) % 8],
                                 isem[(u + 3) % 8])

            pltpu.make_async_copy(h_hbm.at[idxb[s8].at[0]], rows_v[s2],
                                  gsem[s2]).wait()
            pltpu.async_copy(rows_v[s2], acc_sh.at[idxb[s8].at[1]],
                             ssem[s2], add=True)

    for k in (2, 1):
        pltpu.make_async_copy(rows_v[(CPW - k) % 2],
                              acc_sh.at[idxb[0].at[1]],
                              ssem[(CPW - k) % 2]).wait()

    plsc.subcore_barrier()
    pltpu.sync_copy(acc_sh.at[pl.ds(sid * RPT, RPT)],
                    out_hbm.at[cid].at[pl.ds(sid * RPT, RPT)])


_agg_kernel = pl.kernel(
    _agg_body,
    out_type=jax.ShapeDtypeStruct((NC, ROWS, H), jnp.float32),
    mesh=_mesh,
    scratch_types=[
        pltpu.VMEM_SHARED((ROWS, H), jnp.float32),
        [pltpu.VMEM((2, CH), jnp.int32)] * 8,
        [pltpu.VMEM((CH, H), jnp.float32)] * 2,
        [pltpu.SemaphoreType.DMA] * 2,
        [pltpu.SemaphoreType.DMA] * 8,
        [pltpu.SemaphoreType.DMA] * 2,
    ],
)


def _dvec(degp):
    return lax.rsqrt(degp[:, 0:1] + degp[:, 1:2] + 1.0)


def _m1u_body(x_ref, w_ref, o_ref):
    o_ref[...] = lax.dot_general(x_ref[...], w_ref[...],
                                 (((1,), (1,)), ((), ())),
                                 precision=lax.Precision.HIGHEST)


def _scale_body(h_ref, dp_ref, o_ref):
    o_ref[...] = h_ref[...] * _dvec(dp_ref[...])


def _e1m2_body(p_ref, hs_ref, dp_ref, b_ref, w_ref, o_ref):
    d = _dvec(dp_ref[...])
    z = (p_ref[0] + p_ref[1] + hs_ref[...]) * d + b_ref[...]
    z = jnp.maximum(z, 0.0)
    h2 = lax.dot_general(z, w_ref[...], (((1,), (1,)), ((), ())),
                         precision=lax.Precision.HIGHEST)
    o_ref[...] = h2 * d


def _e2_body(p_ref, hs_ref, dp_ref, b_ref, o_ref):
    d = _dvec(dp_ref[...])
    z = (p_ref[0] + p_ref[1] + hs_ref[...]) * d + b_ref[...]
    o_ref[...] = jnp.maximum(z, 0.0)


_row_spec = pl.BlockSpec((BR, H), lambda i: (i, 0))
_w_spec = pl.BlockSpec((H, H), lambda i: (0, 0))
_dp_spec = pl.BlockSpec((BR, 2), lambda i: (i, 0))
_b_spec = pl.BlockSpec((1, H), lambda i: (0, 0))
_p_spec = pl.BlockSpec((NC, BR, H), lambda i: (0, i, 0))
_out_shape = jax.ShapeDtypeStruct((N, H), jnp.float32)
_grid = (N // BR,)

_m1u_kernel = pl.pallas_call(
    _m1u_body, grid=_grid,
    in_specs=[_row_spec, _w_spec],
    out_specs=_row_spec, out_shape=_out_shape)

_scale_kernel = pl.pallas_call(
    _scale_body, grid=_grid,
    in_specs=[_row_spec, _dp_spec],
    out_specs=_row_spec, out_shape=_out_shape)

_e1m2_kernel = pl.pallas_call(
    _e1m2_body, grid=_grid,
    in_specs=[_p_spec, _row_spec, _dp_spec, _b_spec, _w_spec],
    out_specs=_row_spec, out_shape=_out_shape)

_e2_kernel = pl.pallas_call(
    _e2_body, grid=_grid,
    in_specs=[_p_spec, _row_spec, _dp_spec, _b_spec],
    out_specs=_row_spec, out_shape=_out_shape)


def kernel(x, edge_index, W1, b1, W2, b2):
    src = edge_index[0].astype(jnp.int32)
    dst = edge_index[1].astype(jnp.int32)
    npad = PE - E
    pidx = jnp.arange(npad, dtype=jnp.int32)
    src_p = jnp.concatenate([src, pidx % N]).reshape(-1, CH)
    dst_p = jnp.concatenate([dst, N + pidx % (ROWS - N)]).reshape(-1, CH)
    packed = jnp.stack([src_p, dst_p], axis=1)
    b1r = b1.reshape(1, H)
    b2r = b2.reshape(1, H)

    degp = _deg_kernel(dst_p)
    h1u = _m1u_kernel(x, W1)
    degpT = degp.T

    h1s = _scale_kernel(h1u, degpT)
    p1 = _agg_kernel(h1s, packed)
    h2s = _e1m2_kernel(p1, h1s, degpT, b1r, W2)
    p2 = _agg_kernel(h2s, packed)
    return _e2_kernel(p2, h2s, degpT, b2r)


# speedup vs baseline: 1.0024x; 1.0024x over previous
import jax
import jax.numpy as jnp
from jax import lax
from jax.experimental import pallas as pl
from jax.experimental.pallas import tpu as pltpu
from jax.experimental.pallas import tpu_sc as plsc

N = 10000
H = 128
E = 320000
NC = 2
NS = 16
NW = NC * NS
CH = 128
EPW = 10240
PE = EPW * NW
CPW = EPW // CH
ROWS = 10240
RPT = ROWS // NS
CHD = 512
CPWD = EPW // CHD
BR = 2000

_mesh = plsc.VectorSubcoreMesh(core_axis_name="c", subcore_axis_name="s")


def _deg_body(d2_hbm, out_hbm, deg_sh, idxb, ones_v, zb_v, isem):
    cid = lax.axis_index("c")
    sid = lax.axis_index("s")
    wid = sid * NC + cid

    @pl.loop(0, RPT // 16)
    def _(i):
        zb_v[pl.ds(i * 16, 16)] = jnp.zeros((16,), jnp.float32)

    pltpu.sync_copy(zb_v, deg_sh.at[pl.ds(sid * RPT, RPT)])

    @pl.loop(0, CH // 16)
    def _(i):
        ones_v[pl.ds(i * 16, 16)] = jnp.full((16,), 1.0, jnp.float32)

    plsc.subcore_barrier()
    base = wid * (EPW // CH)
    rpc = CHD // CH
    pltpu.async_copy(d2_hbm.at[pl.ds(base, rpc)], idxb[0], isem[0])

    @pl.loop(0, CPWD, step=2)
    def _(c):
        for b in (0, 1):
            cc = c + b

            @pl.when(cc + 1 < CPWD)
            def _():
                pltpu.async_copy(
                    d2_hbm.at[pl.ds(base + (cc + 1) * rpc, rpc)],
                    idxb[b ^ 1], isem[b ^ 1])

            pltpu.make_async_copy(d2_hbm.at[pl.ds(0, rpc)], idxb[b],
                                  isem[b]).wait()
            for j in range(rpc):
                pltpu.sync_copy(ones_v, deg_sh.at[idxb[b].at[j]], add=True)

    plsc.subcore_barrier()
    pltpu.sync_copy(deg_sh.at[pl.ds(sid * RPT, RPT)],
                    out_hbm.at[cid].at[pl.ds(sid * RPT, RPT)])


_deg_kernel = pl.kernel(
    _deg_body,
    out_type=jax.ShapeDtypeStruct((NC, ROWS), jnp.float32),
    mesh=_mesh,
    scratch_types=[
        pltpu.VMEM_SHARED((ROWS,), jnp.float32),
        [pltpu.VMEM((CHD // CH, CH), jnp.int32)] * 2,
        pltpu.VMEM((CH,), jnp.float32),
        pltpu.VMEM((RPT,), jnp.float32),
        [pltpu.SemaphoreType.DMA] * 2,
    ],
)


def _agg_body(h_hbm, pk_hbm, out_hbm, acc_sh, idxb, rows_v, gsem, isem):
    cid = lax.axis_index("c")
    sid = lax.axis_index("s")
    wid = sid * NC + cid

    @pl.loop(0, CH)
    def _(r):
        @pl.loop(0, H // 16)
        def _(j):
            rows_v[0].at[r][pl.ds(j * 16, 16)] = jnp.zeros((16,), jnp.float32)

    @pl.loop(0, RPT // CH)
    def _(i):
        pltpu.sync_copy(rows_v[0], acc_sh.at[pl.ds(sid * RPT + i * CH, CH)])

    plsc.subcore_barrier()
    base = wid * CPW

    for k in range(3):
        pltpu.async_copy(pk_hbm.at[base + k], idxb[k], isem[k])
    pltpu.make_async_copy(pk_hbm.at[0], idxb[0], isem[0]).wait()
    pltpu.async_copy(h_hbm.at[idxb[0].at[0]], rows_v[0], gsem[0])

    @pl.loop(0, CPW, step=4)
    def _(c):
        for u in range(4):
            cc = c + u
            ib = u
            b = u & 1
            nib = (u + 1) % 4

            @pl.when(cc + 1 < CPW)
            def _():
                pltpu.make_async_copy(pk_hbm.at[0], idxb[nib],
                                      isem[nib]).wait()
                pltpu.async_copy(h_hbm.at[idxb[nib].at[0]], rows_v[b ^ 1],
                                 gsem[b ^ 1])

            @pl.when(cc + 3 < CPW)
            def _():
                pltpu.async_copy(pk_hbm.at[base + cc + 3], idxb[(u + 3) % 4],
                                 isem[(u + 3) % 4])

            pltpu.make_async_copy(h_hbm.at[idxb[ib].at[0]], rows_v[b],
                                  gsem[b]).wait()
            pltpu.sync_copy(rows_v[b], acc_sh.at[idxb[ib].at[1]], add=True)

    plsc.subcore_barrier()
    pltpu.sync_copy(acc_sh.at[pl.ds(sid * RPT, RPT)],
                    out_hbm.at[cid].at[pl.ds(sid * RPT, RPT)])


_agg_kernel = pl.kernel(
    _agg_body,
    out_type=jax.ShapeDtypeStruct((NC, ROWS, H), jnp.float32),
    mesh=_mesh,
    scratch_types=[
        pltpu.VMEM_SHARED((ROWS, H), jnp.float32),
        [pltpu.VMEM((2, CH), jnp.int32)] * 4,
        [pltpu.VMEM((CH, H), jnp.float32)] * 2,
        [pltpu.SemaphoreType.DMA] * 2,
        [pltpu.SemaphoreType.DMA] * 4,
    ],
)


def _dvec(degp):
    return lax.rsqrt(degp[:, 0:1] + degp[:, 1:2] + 1.0)


def _m1_body(x_ref, w_ref, dp_ref, o_ref):
    d = _dvec(dp_ref[...])
    h = lax.dot_general(x_ref[...], w_ref[...], (((1,), (1,)), ((), ())),
                        precision=lax.Precision.HIGHEST)
    o_ref[...] = h * d


def _e1m2_body(p_ref, hs_ref, dp_ref, b_ref, w_ref, o_ref):
    d = _dvec(dp_ref[...])
    z = (p_ref[0] + p_ref[1] + hs_ref[...]) * d + b_ref[...]
    z = jnp.maximum(z, 0.0)
    h2 = lax.dot_general(z, w_ref[...], (((1,), (1,)), ((), ())),
                         precision=lax.Precision.HIGHEST)
    o_ref[...] = h2 * d


def _e2_body(p_ref, hs_ref, dp_ref, b_ref, o_ref):
    d = _dvec(dp_ref[...])
    z = (p_ref[0] + p_ref[1] + hs_ref[...]) * d + b_ref[...]
    o_ref[...] = jnp.maximum(z, 0.0)


_row_spec = pl.BlockSpec((BR, H), lambda i: (i, 0))
_w_spec = pl.BlockSpec((H, H), lambda i: (0, 0))
_dp_spec = pl.BlockSpec((BR, 2), lambda i: (i, 0))
_b_spec = pl.BlockSpec((1, H), lambda i: (0, 0))
_p_spec = pl.BlockSpec((NC, BR, H), lambda i: (0, i, 0))
_out_shape = jax.ShapeDtypeStruct((N, H), jnp.float32)
_grid = (N // BR,)

_m1_kernel = pl.pallas_call(
    _m1_body, grid=_grid,
    in_specs=[_row_spec, _w_spec, _dp_spec],
    out_specs=_row_spec, out_shape=_out_shape)

_e1m2_kernel = pl.pallas_call(
    _e1m2_body, grid=_grid,
    in_specs=[_p_spec, _row_spec, _dp_spec, _b_spec, _w_spec],
    out_specs=_row_spec, out_shape=_out_shape)

_e2_kernel = pl.pallas_call(
    _e2_body, grid=_grid,
    in_specs=[_p_spec, _row_spec, _dp_spec, _b_spec],
    out_specs=_row_spec, out_shape=_out_shape)


def kernel(x, edge_index, W1, b1, W2, b2):
    src = edge_index[0].astype(jnp.int32)
    dst = edge_index[1].astype(jnp.int32)
    npad = PE - E
    pidx = jnp.arange(npad, dtype=jnp.int32)
    src_p = jnp.concatenate([src, pidx % N]).reshape(-1, CH)
    dst_p = jnp.concatenate([dst, N + pidx % (ROWS - N)]).reshape(-1, CH)
    packed = jnp.stack([src_p, dst_p], axis=1)
    b1r = b1.reshape(1, H)
    b2r = b2.reshape(1, H)

    degp = _deg_kernel(dst_p)
    degpT = degp.T

    h1s = _m1_kernel(x, W1, degpT)
    p1 = _agg_kernel(h1s, packed)
    h2s = _e1m2_kernel(p1, h1s, degpT, b1r, W2)
    p2 = _agg_kernel(h2s, packed)
    return _e2_kernel(p2, h2s, degpT, b2r)


# speedup vs baseline: 1.0119x; 1.0095x over previous
import jax
import jax.numpy as jnp
from jax import lax
from jax.experimental import pallas as pl
from jax.experimental.pallas import tpu as pltpu
from jax.experimental.pallas import tpu_sc as plsc

N = 10000
H = 128
E = 320000
NC = 2
NS = 16
NW = NC * NS
CH = 128
EPW = 10240
PE = EPW * NW
CPW = EPW // CH
ROWS = 10240
RPT = ROWS // NS
CHD = 1024
CPWD = EPW // CHD
BR = 2000

_mesh = plsc.VectorSubcoreMesh(core_axis_name="c", subcore_axis_name="s")


def _deg_body(d2_hbm, out_hbm, deg_sh, idxb, ones_v, zb_v, isem):
    cid = lax.axis_index("c")
    sid = lax.axis_index("s")
    wid = sid * NC + cid

    @pl.loop(0, RPT // 16)
    def _(i):
        zb_v[pl.ds(i * 16, 16)] = jnp.zeros((16,), jnp.float32)

    pltpu.sync_copy(zb_v, deg_sh.at[pl.ds(sid * RPT, RPT)])

    @pl.loop(0, CH // 16)
    def _(i):
        ones_v[pl.ds(i * 16, 16)] = jnp.full((16,), 1.0, jnp.float32)

    plsc.subcore_barrier()
    base = wid * (EPW // CH)
    rpc = CHD // CH
    pltpu.async_copy(d2_hbm.at[pl.ds(base, rpc)], idxb[0], isem[0])

    @pl.loop(0, CPWD, step=2)
    def _(c):
        for b in (0, 1):
            cc = c + b

            @pl.when(cc + 1 < CPWD)
            def _():
                pltpu.async_copy(
                    d2_hbm.at[pl.ds(base + (cc + 1) * rpc, rpc)],
                    idxb[b ^ 1], isem[b ^ 1])

            pltpu.make_async_copy(d2_hbm.at[pl.ds(0, rpc)], idxb[b],
                                  isem[b]).wait()
            for j in range(rpc):
                pltpu.sync_copy(ones_v, deg_sh.at[idxb[b].at[j]], add=True)

    plsc.subcore_barrier()
    pltpu.sync_copy(deg_sh.at[pl.ds(sid * RPT, RPT)],
                    out_hbm.at[cid].at[pl.ds(sid * RPT, RPT)])


_deg_kernel = pl.kernel(
    _deg_body,
    out_type=jax.ShapeDtypeStruct((NC, ROWS), jnp.float32),
    mesh=_mesh,
    scratch_types=[
        pltpu.VMEM_SHARED((ROWS,), jnp.float32),
        [pltpu.VMEM((CHD // CH, CH), jnp.int32)] * 2,
        pltpu.VMEM((CH,), jnp.float32),
        pltpu.VMEM((RPT,), jnp.float32),
        [pltpu.SemaphoreType.DMA] * 2,
    ],
)


def _agg_body(h_hbm, pk_hbm, out_hbm, acc_sh, idxb, rows_v, gsem, isem):
    cid = lax.axis_index("c")
    sid = lax.axis_index("s")
    wid = sid * NC + cid

    @pl.loop(0, CH)
    def _(r):
        @pl.loop(0, H // 16)
        def _(j):
            rows_v[0].at[r][pl.ds(j * 16, 16)] = jnp.zeros((16,), jnp.float32)

    @pl.loop(0, RPT // CH)
    def _(i):
        pltpu.sync_copy(rows_v[0], acc_sh.at[pl.ds(sid * RPT + i * CH, CH)])

    plsc.subcore_barrier()
    base = wid * CPW

    for k in range(3):
        pltpu.async_copy(pk_hbm.at[base + k], idxb[k], isem[k])
    pltpu.make_async_copy(pk_hbm.at[0], idxb[0], isem[0]).wait()
    pltpu.async_copy(h_hbm.at[idxb[0].at[0]], rows_v[0], gsem[0])

    @pl.loop(0, CPW, step=4)
    def _(c):
        for u in range(4):
            cc = c + u
            ib = u
            b = u & 1
            nib = (u + 1) % 4

            @pl.when(cc + 1 < CPW)
            def _():
                pltpu.make_async_copy(pk_hbm.at[0], idxb[nib],
                                      isem[nib]).wait()
                pltpu.async_copy(h_hbm.at[idxb[nib].at[0]], rows_v[b ^ 1],
                                 gsem[b ^ 1])

            @pl.when(cc + 3 < CPW)
            def _():
                pltpu.async_copy(pk_hbm.at[base + cc + 3], idxb[(u + 3) % 4],
                                 isem[(u + 3) % 4])

            pltpu.make_async_copy(h_hbm.at[idxb[ib].at[0]], rows_v[b],
                                  gsem[b]).wait()
            pltpu.sync_copy(rows_v[b], acc_sh.at[idxb[ib].at[1]], add=True)

    plsc.subcore_barrier()
    pltpu.sync_copy(acc_sh.at[pl.ds(sid * RPT, RPT)],
                    out_hbm.at[cid].at[pl.ds(sid * RPT, RPT)])


_agg_kernel = pl.kernel(
    _agg_body,
    out_type=jax.ShapeDtypeStruct((NC, ROWS, H), jnp.float32),
    mesh=_mesh,
    scratch_types=[
        pltpu.VMEM_SHARED((ROWS, H), jnp.float32),
        [pltpu.VMEM((2, CH), jnp.int32)] * 4,
        [pltpu.VMEM((CH, H), jnp.float32)] * 2,
        [pltpu.SemaphoreType.DMA] * 2,
        [pltpu.SemaphoreType.DMA] * 4,
    ],
)


def _dvec(degp):
    return lax.rsqrt(degp[:, 0:1] + degp[:, 1:2] + 1.0)


def _m1_body(x_ref, w_ref, dp_ref, o_ref):
    d = _dvec(dp_ref[...])
    h = lax.dot_general(x_ref[...], w_ref[...], (((1,), (1,)), ((), ())),
                        precision=lax.Precision.HIGHEST)
    o_ref[...] = h * d


def _e1m2_body(p_ref, hs_ref, dp_ref, b_ref, w_ref, o_ref):
    d = _dvec(dp_ref[...])
    z = (p_ref[0] + p_ref[1] + hs_ref[...]) * d + b_ref[...]
    z = jnp.maximum(z, 0.0)
    h2 = lax.dot_general(z, w_ref[...], (((1,), (1,)), ((), ())),
                         precision=lax.Precision.HIGHEST)
    o_ref[...] = h2 * d


def _e2_body(p_ref, hs_ref, dp_ref, b_ref, o_ref):
    d = _dvec(dp_ref[...])
    z = (p_ref[0] + p_ref[1] + hs_ref[...]) * d + b_ref[...]
    o_ref[...] = jnp.maximum(z, 0.0)


_row_spec = pl.BlockSpec((BR, H), lambda i: (i, 0))
_w_spec = pl.BlockSpec((H, H), lambda i: (0, 0))
_dp_spec = pl.BlockSpec((BR, 2), lambda i: (i, 0))
_b_spec = pl.BlockSpec((1, H), lambda i: (0, 0))
_p_spec = pl.BlockSpec((NC, BR, H), lambda i: (0, i, 0))
_out_shape = jax.ShapeDtypeStruct((N, H), jnp.float32)
_grid = (N // BR,)

_m1_kernel = pl.pallas_call(
    _m1_body, grid=_grid,
    in_specs=[_row_spec, _w_spec, _dp_spec],
    out_specs=_row_spec, out_shape=_out_shape)

_e1m2_kernel = pl.pallas_call(
    _e1m2_body, grid=_grid,
    in_specs=[_p_spec, _row_spec, _dp_spec, _b_spec, _w_spec],
    out_specs=_row_spec, out_shape=_out_shape)

_e2_kernel = pl.pallas_call(
    _e2_body, grid=_grid,
    in_specs=[_p_spec, _row_spec, _dp_spec, _b_spec],
    out_specs=_row_spec, out_shape=_out_shape)


def kernel(x, edge_index, W1, b1, W2, b2):
    src = edge_index[0].astype(jnp.int32)
    dst = edge_index[1].astype(jnp.int32)
    npad = PE - E
    pidx = jnp.arange(npad, dtype=jnp.int32)
    src_p = jnp.concatenate([src, pidx % N]).reshape(-1, CH)
    dst_p = jnp.concatenate([dst, N + pidx % (ROWS - N)]).reshape(-1, CH)
    packed = jnp.stack([src_p, dst_p], axis=1)
    b1r = b1.reshape(1, H)
    b2r = b2.reshape(1, H)

    degp = _deg_kernel(dst_p)
    degpT = degp.T

    h1s = _m1_kernel(x, W1, degpT)
    p1 = _agg_kernel(h1s, packed)
    h2s = _e1m2_kernel(p1, h1s, degpT, b1r, W2)
    p2 = _agg_kernel(h2s, packed)
    return _e2_kernel(p2, h2s, degpT, b2r)
